# Initial kernel scaffold; baseline (speedup 1.0000x reference)
import functools

import jax
import jax.numpy as jnp
from jax import lax
from jax.experimental import pallas as pl
from jax.experimental.pallas import tpu as pltpu
from jax.experimental.pallas import tpu_sc as plsc

_NC = 2
_NS = 16
_LANES = 16


def _ceil_to(a, m):
    return (a + m - 1) // m * m


def _sc_degree(dst2d, nh):
    rows = dst2d.shape[0]
    half = rows // _NC
    per_tile = half // _NS
    sl = nh // _NS
    mesh = plsc.VectorSubcoreMesh(core_axis_name="c", subcore_axis_name="s")

    @functools.partial(
        pl.kernel,
        out_type=jax.ShapeDtypeStruct((_NC, nh), jnp.float32),
        mesh=mesh,
        scratch_types=[
            pltpu.VMEM((per_tile, 128), jnp.int32),
            pltpu.VMEM((nh,), jnp.float32),
            pltpu.VMEM((sl,), jnp.float32),
            pltpu.VMEM((sl,), jnp.float32),
            pltpu.VMEM_SHARED((_NS, nh), jnp.float32),
        ],
    )
    def k(dst_hbm, deg_hbm, dst_v, hist_v, acc_v, buf_v, part_sp):
        c = lax.axis_index("c")
        s = lax.axis_index("s")
        zero16 = jnp.zeros((_LANES,), jnp.float32)
        one16 = jnp.ones((_LANES,), jnp.float32)

        @pl.loop(0, nh, step=_LANES)
        def _(i):
            hist_v[pl.ds(pl.multiple_of(i, _LANES), _LANES)] = zero16

        pltpu.sync_copy(dst_hbm.at[pl.ds(c * half + s * per_tile, per_tile)],
                        dst_v)

        @pl.loop(0, per_tile)
        def _(r):
            for j in range(128 // _LANES):
                idx = dst_v[r, pl.ds(j * _LANES, _LANES)]
                plsc.addupdate_scatter(hist_v, [idx], one16)

        pltpu.sync_copy(hist_v, part_sp.at[s])
        plsc.subcore_barrier()

        pltpu.sync_copy(part_sp.at[0, pl.ds(s * sl, sl)], acc_v)
        for t in range(1, _NS):
            pltpu.sync_copy(part_sp.at[t, pl.ds(s * sl, sl)], buf_v)

            @pl.loop(0, sl, step=_LANES)
            def _(j):
                jj = pl.multiple_of(j, _LANES)
                acc_v[pl.ds(jj, _LANES)] = (
                    acc_v[pl.ds(jj, _LANES)] + buf_v[pl.ds(jj, _LANES)])

        @pl.when(c == 0)
        def _():
            pltpu.sync_copy(acc_v, deg_hbm.at[0, pl.ds(s * sl, sl)])

        @pl.when(c == 1)
        def _():
            pltpu.sync_copy(acc_v, deg_hbm.at[1, pl.ds(s * sl, sl)])

    return k(dst2d)


def _sc_aggregate(g0, g1, src2d, dst2d, nh):
    rows = src2d.shape[0]
    per_tile = rows // _NS
    zrows = nh // _NS
    mesh = plsc.VectorSubcoreMesh(core_axis_name="c", subcore_axis_name="s")

    @functools.partial(
        pl.kernel,
        out_type=[jax.ShapeDtypeStruct((nh, 128), jnp.float32)] * 2,
        mesh=mesh,
        scratch_types=[
            pltpu.VMEM((per_tile, 128), jnp.int32),
            pltpu.VMEM((per_tile, 128), jnp.int32),
            pltpu.VMEM((128, 128), jnp.float32),
            pltpu.VMEM_SHARED((nh, 128), jnp.float32),
        ],
    )
    def k(g0_hbm, g1_hbm, src_hbm, dst_hbm, out0_hbm, out1_hbm,
          src_v, dst_v, rows_v, acc_sp):
        c = lax.axis_index("c")
        s = lax.axis_index("s")
        zero16 = jnp.zeros((_LANES,), jnp.float32)

        @pl.loop(0, 128)
        def _(i):
            for j in range(128 // _LANES):
                rows_v[i, pl.ds(j * _LANES, _LANES)] = zero16

        for j in range(zrows // 128):
            pltpu.sync_copy(rows_v,
                            acc_sp.at[pl.ds(s * zrows + j * 128, 128)])

        pltpu.sync_copy(src_hbm.at[pl.ds(s * per_tile, per_tile)], src_v)
        pltpu.sync_copy(dst_hbm.at[pl.ds(s * per_tile, per_tile)], dst_v)
        plsc.subcore_barrier()

        def run_half(g_hbm):
            @pl.loop(0, per_tile)
            def _(kk):
                pltpu.sync_copy(g_hbm.at[src_v.at[kk]], rows_v)
                pltpu.sync_copy(rows_v, acc_sp.at[dst_v.at[kk]], add=True)

        @pl.when(c == 0)
        def _():
            run_half(g0_hbm)

        @pl.when(c == 1)
        def _():
            run_half(g1_hbm)

        plsc.subcore_barrier()
        out_slc = pl.ds(s * zrows, zrows)

        @pl.when(c == 0)
        def _():
            pltpu.sync_copy(acc_sp.at[out_slc], out0_hbm.at[out_slc])

        @pl.when(c == 1)
        def _():
            pltpu.sync_copy(acc_sp.at[out_slc], out1_hbm.at[out_slc])

    return k(g0, g1, src2d, dst2d)


def _tc_layer1(x, w, p0, p1, grid, rb):
    n, d = x.shape

    def body(x_ref, w_ref, p0_ref, p1_ref, h_ref, g0_ref, g1_ref):
        h = jnp.dot(x_ref[...], w_ref[...],
                    preferred_element_type=jnp.float32,
                    precision=lax.Precision.HIGHEST)
        deg = p0_ref[...] + p1_ref[...] + 1.0
        dis = lax.rsqrt(deg)
        g = h * dis
        h_ref[...] = h
        g0_ref[...] = g[:, :d // 2]
        g1_ref[...] = g[:, d // 2:]

    return pl.pallas_call(
        body,
        grid=(grid,),
        in_specs=[
            pl.BlockSpec((rb, d), lambda i: (i, 0)),
            pl.BlockSpec((d, d), lambda i: (0, 0)),
            pl.BlockSpec((rb, 1), lambda i: (i, 0)),
            pl.BlockSpec((rb, 1), lambda i: (i, 0)),
        ],
        out_specs=[
            pl.BlockSpec((rb, d), lambda i: (i, 0)),
            pl.BlockSpec((rb, d // 2), lambda i: (i, 0)),
            pl.BlockSpec((rb, d // 2), lambda i: (i, 0)),
        ],
        out_shape=[
            jax.ShapeDtypeStruct((n, d), jnp.float32),
            jax.ShapeDtypeStruct((n, d // 2), jnp.float32),
            jax.ShapeDtypeStruct((n, d // 2), jnp.float32),
        ],
    )(x, w, p0, p1)


def _tc_layer2(a0, a1, h_prev, p0, p1, b, w, grid, rb):
    n = h_prev.shape[0]
    d = h_prev.shape[1]

    def body(a0_ref, a1_ref, h_ref, p0_ref, p1_ref, b_ref, w_ref,
             h2_ref, g0_ref, g1_ref):
        deg = p0_ref[...] + p1_ref[...] + 1.0
        dis = lax.rsqrt(deg)
        inv = 1.0 / deg
        agg = jnp.concatenate([a0_ref[...], a1_ref[...]], axis=1)
        act = jnp.maximum(agg * dis + h_ref[...] * inv + b_ref[...], 0.0)
        h2 = jnp.dot(act, w_ref[...],
                     preferred_element_type=jnp.float32,
                     precision=lax.Precision.HIGHEST)
        g2 = h2 * dis
        h2_ref[...] = h2
        g0_ref[...] = g2[:, :d // 2]
        g1_ref[...] = g2[:, d // 2:]

    return pl.pallas_call(
        body,
        grid=(grid,),
        in_specs=[
            pl.BlockSpec((rb, d // 2), lambda i: (i, 0)),
            pl.BlockSpec((rb, d // 2), lambda i: (i, 0)),
            pl.BlockSpec((rb, d), lambda i: (i, 0)),
            pl.BlockSpec((rb, 1), lambda i: (i, 0)),
            pl.BlockSpec((rb, 1), lambda i: (i, 0)),
            pl.BlockSpec((1, d), lambda i: (0, 0)),
            pl.BlockSpec((d, d), lambda i: (0, 0)),
        ],
        out_specs=[
            pl.BlockSpec((rb, d), lambda i: (i, 0)),
            pl.BlockSpec((rb, d // 2), lambda i: (i, 0)),
            pl.BlockSpec((rb, d // 2), lambda i: (i, 0)),
        ],
        out_shape=[
            jax.ShapeDtypeStruct((n, d), jnp.float32),
            jax.ShapeDtypeStruct((n, d // 2), jnp.float32),
            jax.ShapeDtypeStruct((n, d // 2), jnp.float32),
        ],
    )(a0, a1, h_prev, p0, p1, b, w)


def _tc_final(a0, a1, h_prev, p0, p1, b, grid, rb):
    n = h_prev.shape[0]
    d = h_prev.shape[1]

    def body(a0_ref, a1_ref, h_ref, p0_ref, p1_ref, b_ref, o_ref):
        deg = p0_ref[...] + p1_ref[...] + 1.0
        dis = lax.rsqrt(deg)
        inv = 1.0 / deg
        agg = jnp.concatenate([a0_ref[...], a1_ref[...]], axis=1)
        o_ref[...] = agg * dis + h_ref[...] * inv + b_ref[...]

    return pl.pallas_call(
        body,
        grid=(grid,),
        in_specs=[
            pl.BlockSpec((rb, d // 2), lambda i: (i, 0)),
            pl.BlockSpec((rb, d // 2), lambda i: (i, 0)),
            pl.BlockSpec((rb, d), lambda i: (i, 0)),
            pl.BlockSpec((rb, 1), lambda i: (i, 0)),
            pl.BlockSpec((rb, 1), lambda i: (i, 0)),
            pl.BlockSpec((1, d), lambda i: (0, 0)),
        ],
        out_specs=pl.BlockSpec((rb, d), lambda i: (i, 0)),
        out_shape=jax.ShapeDtypeStruct((n, d), jnp.float32),
    )(a0, a1, h_prev, p0, p1, b)


@jax.jit
def kernel(x, edge_index, W1, b1, W2, b2):
    n, d = x.shape
    e = edge_index.shape[1]

    epad = _ceil_to(e, _NC * _NS * 128)
    src_p = jnp.concatenate(
        [edge_index[0], jnp.zeros((epad - e,), jnp.int32)]).reshape(-1, 128)
    dst_p = jnp.concatenate(
        [edge_index[1], jnp.full((epad - e,), n, jnp.int32)]).reshape(-1, 128)

    nh = _ceil_to(n + 1, _NS * 128)

    deg_parts = _sc_degree(dst_p, nh)
    p0 = deg_parts[0, :n].reshape(n, 1)
    p1 = deg_parts[1, :n].reshape(n, 1)

    grid, rb = 10, n // 10
    b1r = b1.reshape(1, d)
    b2r = b2.reshape(1, d)

    h1, g1a, g1b = _tc_layer1(x, W1, p0, p1, grid, rb)
    agg1a, agg1b = _sc_aggregate(g1a, g1b, src_p, dst_p, nh)
    h2, g2a, g2b = _tc_layer2(agg1a[:n], agg1b[:n], h1, p0, p1, b1r, W2,
                              grid, rb)
    agg2a, agg2b = _sc_aggregate(g2a, g2b, src_p, dst_p, nh)
    out = _tc_final(agg2a[:n], agg2b[:n], h2, p0, p1, b2r, grid, rb)
    return out


# speedup vs baseline: 7.2717x; 7.2717x over previous
import dataclasses
import functools

import jax
import jax.numpy as jnp
from jax import lax
from jax.experimental import pallas as pl
from jax.experimental.pallas import tpu as pltpu
from jax.experimental.pallas import tpu_sc as plsc

_NC = 2
_NS = 16
_LANES = 16


def _sc_compiler_params():
    cp = pltpu.CompilerParams()
    if "needs_layout_passes" in pltpu.CompilerParams.__dataclass_fields__:
        cp = dataclasses.replace(cp, needs_layout_passes=False)
    return cp


def _ceil_to(a, m):
    return (a + m - 1) // m * m


def _sc_degree(dst2d, nh):
    rows = dst2d.shape[0]
    half = rows // _NC
    per_tile = half // _NS
    sl = nh // _NS
    mesh = plsc.VectorSubcoreMesh(core_axis_name="c", subcore_axis_name="s")

    @functools.partial(
        pl.kernel,
        out_type=jax.ShapeDtypeStruct((_NC, nh), jnp.float32),
        mesh=mesh,
        scratch_types=[
            pltpu.VMEM((per_tile, 128), jnp.int32),
            pltpu.VMEM((nh,), jnp.float32),
            pltpu.VMEM((sl,), jnp.float32),
            pltpu.VMEM((sl,), jnp.float32),
            pltpu.VMEM_SHARED((_NS, nh), jnp.float32),
        ],
        compiler_params=_sc_compiler_params(),
    )
    def k(dst_hbm, deg_hbm, dst_v, hist_v, acc_v, buf_v, part_sp):
        c = lax.axis_index("c")
        s = lax.axis_index("s")
        zero16 = jnp.zeros((_LANES,), jnp.float32)
        one16 = jnp.ones((_LANES,), jnp.float32)

        @pl.loop(0, nh, step=_LANES)
        def _(i):
            hist_v[pl.ds(pl.multiple_of(i, _LANES), _LANES)] = zero16

        pltpu.sync_copy(dst_hbm.at[pl.ds(c * half + s * per_tile, per_tile)],
                        dst_v)

        @pl.loop(0, per_tile)
        def _(r):
            for j in range(128 // _LANES):
                idx = dst_v[r, pl.ds(j * _LANES, _LANES)]
                plsc.addupdate_scatter(hist_v, [idx], one16)

        pltpu.sync_copy(hist_v, part_sp.at[s])
        plsc.subcore_barrier()

        pltpu.sync_copy(part_sp.at[0, pl.ds(s * sl, sl)], acc_v)
        for t in range(1, _NS):
            pltpu.sync_copy(part_sp.at[t, pl.ds(s * sl, sl)], buf_v)

            @pl.loop(0, sl, step=_LANES)
            def _(j):
                jj = pl.multiple_of(j, _LANES)
                acc_v[pl.ds(jj, _LANES)] = (
                    acc_v[pl.ds(jj, _LANES)] + buf_v[pl.ds(jj, _LANES)])

        @pl.when(c == 0)
        def _():
            pltpu.sync_copy(acc_v, deg_hbm.at[0, pl.ds(s * sl, sl)])

        @pl.when(c == 1)
        def _():
            pltpu.sync_copy(acc_v, deg_hbm.at[1, pl.ds(s * sl, sl)])

    return k(dst2d)


def _sc_aggregate(g0, g1, src2d, dst2d, nh):
    rows = src2d.shape[0]
    per_tile = rows // _NS
    zrows = nh // _NS
    mesh = plsc.VectorSubcoreMesh(core_axis_name="c", subcore_axis_name="s")

    @functools.partial(
        pl.kernel,
        out_type=[jax.ShapeDtypeStruct((nh, 128), jnp.float32)] * 2,
        mesh=mesh,
        scratch_types=[
            pltpu.VMEM((per_tile, 128), jnp.int32),
            pltpu.VMEM((per_tile, 128), jnp.int32),
            pltpu.VMEM((128, 128), jnp.float32),
            pltpu.VMEM_SHARED((nh, 128), jnp.float32),
        ],
    )
    def k(g0_hbm, g1_hbm, src_hbm, dst_hbm, out0_hbm, out1_hbm,
          src_v, dst_v, rows_v, acc_sp):
        c = lax.axis_index("c")
        s = lax.axis_index("s")
        zero16 = jnp.zeros((_LANES,), jnp.float32)

        @pl.loop(0, 128)
        def _(i):
            for j in range(128 // _LANES):
                rows_v[i, pl.ds(j * _LANES, _LANES)] = zero16

        for j in range(zrows // 128):
            pltpu.sync_copy(rows_v,
                            acc_sp.at[pl.ds(s * zrows + j * 128, 128)])

        pltpu.sync_copy(src_hbm.at[pl.ds(s * per_tile, per_tile)], src_v)
        pltpu.sync_copy(dst_hbm.at[pl.ds(s * per_tile, per_tile)], dst_v)
        plsc.subcore_barrier()

        def run_half(g_hbm):
            @pl.loop(0, per_tile)
            def _(kk):
                pltpu.sync_copy(g_hbm.at[src_v.at[kk]], rows_v)
                pltpu.sync_copy(rows_v, acc_sp.at[dst_v.at[kk]], add=True)

        @pl.when(c == 0)
        def _():
            run_half(g0_hbm)

        @pl.when(c == 1)
        def _():
            run_half(g1_hbm)

        plsc.subcore_barrier()
        out_slc = pl.ds(s * zrows, zrows)

        @pl.when(c == 0)
        def _():
            pltpu.sync_copy(acc_sp.at[out_slc], out0_hbm.at[out_slc])

        @pl.when(c == 1)
        def _():
            pltpu.sync_copy(acc_sp.at[out_slc], out1_hbm.at[out_slc])

    return k(g0, g1, src2d, dst2d)


def _tc_layer1(x, w, p0, p1, grid, rb):
    n, d = x.shape

    def body(x_ref, w_ref, p0_ref, p1_ref, h_ref, g0_ref, g1_ref):
        h = jnp.dot(x_ref[...], w_ref[...],
                    preferred_element_type=jnp.float32,
                    precision=lax.Precision.HIGHEST)
        deg = p0_ref[...] + p1_ref[...] + 1.0
        dis = lax.rsqrt(deg)
        g = h * dis
        h_ref[...] = h
        g0_ref[...] = g[:, :d // 2]
        g1_ref[...] = g[:, d // 2:]

    return pl.pallas_call(
        body,
        grid=(grid,),
        in_specs=[
            pl.BlockSpec((rb, d), lambda i: (i, 0)),
            pl.BlockSpec((d, d), lambda i: (0, 0)),
            pl.BlockSpec((rb, 1), lambda i: (i, 0)),
            pl.BlockSpec((rb, 1), lambda i: (i, 0)),
        ],
        out_specs=[
            pl.BlockSpec((rb, d), lambda i: (i, 0)),
            pl.BlockSpec((rb, d // 2), lambda i: (i, 0)),
            pl.BlockSpec((rb, d // 2), lambda i: (i, 0)),
        ],
        out_shape=[
            jax.ShapeDtypeStruct((n, d), jnp.float32),
            jax.ShapeDtypeStruct((n, d // 2), jnp.float32),
            jax.ShapeDtypeStruct((n, d // 2), jnp.float32),
        ],
    )(x, w, p0, p1)


def _tc_layer2(a0, a1, h_prev, p0, p1, b, w, grid, rb):
    n = h_prev.shape[0]
    d = h_prev.shape[1]

    def body(a0_ref, a1_ref, h_ref, p0_ref, p1_ref, b_ref, w_ref,
             h2_ref, g0_ref, g1_ref):
        deg = p0_ref[...] + p1_ref[...] + 1.0
        dis = lax.rsqrt(deg)
        inv = 1.0 / deg
        agg = jnp.concatenate([a0_ref[...], a1_ref[...]], axis=1)
        act = jnp.maximum(agg * dis + h_ref[...] * inv + b_ref[...], 0.0)
        h2 = jnp.dot(act, w_ref[...],
                     preferred_element_type=jnp.float32,
                     precision=lax.Precision.HIGHEST)
        g2 = h2 * dis
        h2_ref[...] = h2
        g0_ref[...] = g2[:, :d // 2]
        g1_ref[...] = g2[:, d // 2:]

    return pl.pallas_call(
        body,
        grid=(grid,),
        in_specs=[
            pl.BlockSpec((rb, d // 2), lambda i: (i, 0)),
            pl.BlockSpec((rb, d // 2), lambda i: (i, 0)),
            pl.BlockSpec((rb, d), lambda i: (i, 0)),
            pl.BlockSpec((rb, 1), lambda i: (i, 0)),
            pl.BlockSpec((rb, 1), lambda i: (i, 0)),
            pl.BlockSpec((1, d), lambda i: (0, 0)),
            pl.BlockSpec((d, d), lambda i: (0, 0)),
        ],
        out_specs=[
            pl.BlockSpec((rb, d), lambda i: (i, 0)),
            pl.BlockSpec((rb, d // 2), lambda i: (i, 0)),
            pl.BlockSpec((rb, d // 2), lambda i: (i, 0)),
        ],
        out_shape=[
            jax.ShapeDtypeStruct((n, d), jnp.float32),
            jax.ShapeDtypeStruct((n, d // 2), jnp.float32),
            jax.ShapeDtypeStruct((n, d // 2), jnp.float32),
        ],
    )(a0, a1, h_prev, p0, p1, b, w)


def _tc_final(a0, a1, h_prev, p0, p1, b, grid, rb):
    n = h_prev.shape[0]
    d = h_prev.shape[1]

    def body(a0_ref, a1_ref, h_ref, p0_ref, p1_ref, b_ref, o_ref):
        deg = p0_ref[...] + p1_ref[...] + 1.0
        dis = lax.rsqrt(deg)
        inv = 1.0 / deg
        agg = jnp.concatenate([a0_ref[...], a1_ref[...]], axis=1)
        o_ref[...] = agg * dis + h_ref[...] * inv + b_ref[...]

    return pl.pallas_call(
        body,
        grid=(grid,),
        in_specs=[
            pl.BlockSpec((rb, d // 2), lambda i: (i, 0)),
            pl.BlockSpec((rb, d // 2), lambda i: (i, 0)),
            pl.BlockSpec((rb, d), lambda i: (i, 0)),
            pl.BlockSpec((rb, 1), lambda i: (i, 0)),
            pl.BlockSpec((rb, 1), lambda i: (i, 0)),
            pl.BlockSpec((1, d), lambda i: (0, 0)),
        ],
        out_specs=pl.BlockSpec((rb, d), lambda i: (i, 0)),
        out_shape=jax.ShapeDtypeStruct((n, d), jnp.float32),
    )(a0, a1, h_prev, p0, p1, b)


@jax.jit
def kernel(x, edge_index, W1, b1, W2, b2):
    n, d = x.shape
    e = edge_index.shape[1]

    epad = _ceil_to(e, _NC * _NS * 128)
    src_p = jnp.concatenate(
        [edge_index[0], jnp.zeros((epad - e,), jnp.int32)]).reshape(-1, 128)
    dst_p = jnp.concatenate(
        [edge_index[1], jnp.full((epad - e,), n, jnp.int32)]).reshape(-1, 128)

    nh = _ceil_to(n + 1, _NS * 128)

    deg_parts = _sc_degree(dst_p, nh)
    p0 = deg_parts[0, :n].reshape(n, 1)
    p1 = deg_parts[1, :n].reshape(n, 1)

    grid, rb = 10, n // 10
    b1r = b1.reshape(1, d)
    b2r = b2.reshape(1, d)

    h1, g1a, g1b = _tc_layer1(x, W1, p0, p1, grid, rb)
    agg1a, agg1b = _sc_aggregate(g1a, g1b, src_p, dst_p, nh)
    h2, g2a, g2b = _tc_layer2(agg1a[:n], agg1b[:n], h1, p0, p1, b1r, W2,
                              grid, rb)
    agg2a, agg2b = _sc_aggregate(g2a, g2b, src_p, dst_p, nh)
    out = _tc_final(agg2a[:n], agg2b[:n], h2, p0, p1, b2r, grid, rb)
    return out


# speedup vs baseline: 7.6183x; 1.0477x over previous
import dataclasses
import functools

import jax
import jax.numpy as jnp
from jax import lax
from jax.experimental import pallas as pl
from jax.experimental.pallas import tpu as pltpu
from jax.experimental.pallas import tpu_sc as plsc

_NC = 2
_NS = 16
_LANES = 16


def _sc_compiler_params():
    cp = pltpu.CompilerParams()
    if "needs_layout_passes" in pltpu.CompilerParams.__dataclass_fields__:
        cp = dataclasses.replace(cp, needs_layout_passes=False)
    return cp


def _ceil_to(a, m):
    return (a + m - 1) // m * m


def _sc_degree(dst2d, nh):
    rows = dst2d.shape[0]
    half = rows // _NC
    per_tile = half // _NS
    sl = nh // _NS
    mesh = plsc.VectorSubcoreMesh(core_axis_name="c", subcore_axis_name="s")

    @functools.partial(
        pl.kernel,
        out_type=jax.ShapeDtypeStruct((_NC, nh), jnp.float32),
        mesh=mesh,
        scratch_types=[
            pltpu.VMEM((per_tile, 128), jnp.int32),
            pltpu.VMEM((nh,), jnp.float32),
            pltpu.VMEM((sl,), jnp.float32),
            pltpu.VMEM((sl,), jnp.float32),
            pltpu.VMEM_SHARED((_NS, nh), jnp.float32),
        ],
        compiler_params=_sc_compiler_params(),
    )
    def k(dst_hbm, deg_hbm, dst_v, hist_v, acc_v, buf_v, part_sp):
        c = lax.axis_index("c")
        s = lax.axis_index("s")
        zero16 = jnp.zeros((_LANES,), jnp.float32)
        one16 = jnp.ones((_LANES,), jnp.float32)

        @pl.loop(0, nh, step=_LANES)
        def _(i):
            hist_v[pl.ds(pl.multiple_of(i, _LANES), _LANES)] = zero16

        pltpu.sync_copy(dst_hbm.at[pl.ds(c * half + s * per_tile, per_tile)],
                        dst_v)

        @pl.loop(0, per_tile)
        def _(r):
            for j in range(128 // _LANES):
                idx = dst_v[r, pl.ds(j * _LANES, _LANES)]
                plsc.addupdate_scatter(hist_v, [idx], one16)

        pltpu.sync_copy(hist_v, part_sp.at[s])
        plsc.subcore_barrier()

        pltpu.sync_copy(part_sp.at[0, pl.ds(s * sl, sl)], acc_v)
        for t in range(1, _NS):
            pltpu.sync_copy(part_sp.at[t, pl.ds(s * sl, sl)], buf_v)

            @pl.loop(0, sl, step=_LANES)
            def _(j):
                jj = pl.multiple_of(j, _LANES)
                acc_v[pl.ds(jj, _LANES)] = (
                    acc_v[pl.ds(jj, _LANES)] + buf_v[pl.ds(jj, _LANES)])

        @pl.when(c == 0)
        def _():
            pltpu.sync_copy(acc_v, deg_hbm.at[0, pl.ds(s * sl, sl)])

        @pl.when(c == 1)
        def _():
            pltpu.sync_copy(acc_v, deg_hbm.at[1, pl.ds(s * sl, sl)])

    return k(dst2d)


def _sc_aggregate(g0, g1, src2d, dst2d, nh):
    rows = src2d.shape[0]
    per_tile = rows // _NS
    nch = 2
    chunk = per_tile // nch
    zrows = nh // _NS
    mesh = plsc.VectorSubcoreMesh(core_axis_name="c", subcore_axis_name="s")

    @functools.partial(
        pl.kernel,
        out_type=[jax.ShapeDtypeStruct((nh, 128), jnp.float32)] * 2,
        mesh=mesh,
        scratch_types=[
            pltpu.VMEM((chunk, 128), jnp.int32),
            pltpu.VMEM((chunk, 128), jnp.int32),
            pltpu.VMEM((128, 128), jnp.float32),
            pltpu.VMEM((128, 128), jnp.float32),
            pltpu.VMEM_SHARED((nh, 128), jnp.float32),
            pltpu.SemaphoreType.DMA((2,)),
            pltpu.SemaphoreType.DMA((2,)),
        ],
    )
    def k(g0_hbm, g1_hbm, src_hbm, dst_hbm, out0_hbm, out1_hbm,
          src_v, dst_v, b0, b1, acc_sp, gsem, ssem):
        c = lax.axis_index("c")
        s = lax.axis_index("s")
        zero16 = jnp.zeros((_LANES,), jnp.float32)

        @pl.loop(0, 128)
        def _(i):
            for j in range(128 // _LANES):
                b0[i, pl.ds(j * _LANES, _LANES)] = zero16

        for j in range(zrows // 128):
            pltpu.sync_copy(b0, acc_sp.at[pl.ds(s * zrows + j * 128, 128)])

        plsc.subcore_barrier()

        def run_half(g_hbm):
            for ch in range(nch):
                base = s * per_tile + ch * chunk
                pltpu.sync_copy(src_hbm.at[pl.ds(base, chunk)], src_v)
                pltpu.sync_copy(dst_hbm.at[pl.ds(base, chunk)], dst_v)

                @pl.loop(0, chunk // 2)
                def _(t):
                    k0 = 2 * t
                    cp0 = pltpu.async_copy(g_hbm.at[src_v.at[k0]], b0,
                                           gsem.at[0])
                    cp1 = pltpu.async_copy(g_hbm.at[src_v.at[k0 + 1]], b1,
                                           gsem.at[1])
                    cp0.wait()
                    sc0 = pltpu.async_copy(b0, acc_sp.at[dst_v.at[k0]],
                                           ssem.at[0], add=True)
                    cp1.wait()
                    sc1 = pltpu.async_copy(b1, acc_sp.at[dst_v.at[k0 + 1]],
                                           ssem.at[1], add=True)
                    sc0.wait()
                    sc1.wait()

        @pl.when(c == 0)
        def _():
            run_half(g0_hbm)

        @pl.when(c == 1)
        def _():
            run_half(g1_hbm)

        plsc.subcore_barrier()
        out_slc = pl.ds(s * zrows, zrows)

        @pl.when(c == 0)
        def _():
            pltpu.sync_copy(acc_sp.at[out_slc], out0_hbm.at[out_slc])

        @pl.when(c == 1)
        def _():
            pltpu.sync_copy(acc_sp.at[out_slc], out1_hbm.at[out_slc])

    return k(g0, g1, src2d, dst2d)


def _tc_layer1(x, w, p0, p1, grid, rb):
    n, d = x.shape

    def body(x_ref, w_ref, p0_ref, p1_ref, h_ref, g0_ref, g1_ref):
        h = jnp.dot(x_ref[...], w_ref[...],
                    preferred_element_type=jnp.float32,
                    precision=lax.Precision.HIGHEST)
        deg = p0_ref[...] + p1_ref[...] + 1.0
        dis = lax.rsqrt(deg)
        g = h * dis
        h_ref[...] = h
        g0_ref[...] = g[:, :d // 2]
        g1_ref[...] = g[:, d // 2:]

    return pl.pallas_call(
        body,
        grid=(grid,),
        in_specs=[
            pl.BlockSpec((rb, d), lambda i: (i, 0)),
            pl.BlockSpec((d, d), lambda i: (0, 0)),
            pl.BlockSpec((rb, 1), lambda i: (i, 0)),
            pl.BlockSpec((rb, 1), lambda i: (i, 0)),
        ],
        out_specs=[
            pl.BlockSpec((rb, d), lambda i: (i, 0)),
            pl.BlockSpec((rb, d // 2), lambda i: (i, 0)),
            pl.BlockSpec((rb, d // 2), lambda i: (i, 0)),
        ],
        out_shape=[
            jax.ShapeDtypeStruct((n, d), jnp.float32),
            jax.ShapeDtypeStruct((n, d // 2), jnp.float32),
            jax.ShapeDtypeStruct((n, d // 2), jnp.float32),
        ],
    )(x, w, p0, p1)


def _tc_layer2(a0, a1, h_prev, p0, p1, b, w, grid, rb):
    n = h_prev.shape[0]
    d = h_prev.shape[1]

    def body(a0_ref, a1_ref, h_ref, p0_ref, p1_ref, b_ref, w_ref,
             h2_ref, g0_ref, g1_ref):
        deg = p0_ref[...] + p1_ref[...] + 1.0
        dis = lax.rsqrt(deg)
        inv = 1.0 / deg
        agg = jnp.concatenate([a0_ref[...], a1_ref[...]], axis=1)
        act = jnp.maximum(agg * dis + h_ref[...] * inv + b_ref[...], 0.0)
        h2 = jnp.dot(act, w_ref[...],
                     preferred_element_type=jnp.float32,
                     precision=lax.Precision.HIGHEST)
        g2 = h2 * dis
        h2_ref[...] = h2
        g0_ref[...] = g2[:, :d // 2]
        g1_ref[...] = g2[:, d // 2:]

    return pl.pallas_call(
        body,
        grid=(grid,),
        in_specs=[
            pl.BlockSpec((rb, d // 2), lambda i: (i, 0)),
            pl.BlockSpec((rb, d // 2), lambda i: (i, 0)),
            pl.BlockSpec((rb, d), lambda i: (i, 0)),
            pl.BlockSpec((rb, 1), lambda i: (i, 0)),
            pl.BlockSpec((rb, 1), lambda i: (i, 0)),
            pl.BlockSpec((1, d), lambda i: (0, 0)),
            pl.BlockSpec((d, d), lambda i: (0, 0)),
        ],
        out_specs=[
            pl.BlockSpec((rb, d), lambda i: (i, 0)),
            pl.BlockSpec((rb, d // 2), lambda i: (i, 0)),
            pl.BlockSpec((rb, d // 2), lambda i: (i, 0)),
        ],
        out_shape=[
            jax.ShapeDtypeStruct((n, d), jnp.float32),
            jax.ShapeDtypeStruct((n, d // 2), jnp.float32),
            jax.ShapeDtypeStruct((n, d // 2), jnp.float32),
        ],
    )(a0, a1, h_prev, p0, p1, b, w)


def _tc_final(a0, a1, h_prev, p0, p1, b, grid, rb):
    n = h_prev.shape[0]
    d = h_prev.shape[1]

    def body(a0_ref, a1_ref, h_ref, p0_ref, p1_ref, b_ref, o_ref):
        deg = p0_ref[...] + p1_ref[...] + 1.0
        dis = lax.rsqrt(deg)
        inv = 1.0 / deg
        agg = jnp.concatenate([a0_ref[...], a1_ref[...]], axis=1)
        o_ref[...] = agg * dis + h_ref[...] * inv + b_ref[...]

    return pl.pallas_call(
        body,
        grid=(grid,),
        in_specs=[
            pl.BlockSpec((rb, d // 2), lambda i: (i, 0)),
            pl.BlockSpec((rb, d // 2), lambda i: (i, 0)),
            pl.BlockSpec((rb, d), lambda i: (i, 0)),
            pl.BlockSpec((rb, 1), lambda i: (i, 0)),
            pl.BlockSpec((rb, 1), lambda i: (i, 0)),
            pl.BlockSpec((1, d), lambda i: (0, 0)),
        ],
        out_specs=pl.BlockSpec((rb, d), lambda i: (i, 0)),
        out_shape=jax.ShapeDtypeStruct((n, d), jnp.float32),
    )(a0, a1, h_prev, p0, p1, b)


@jax.jit
def kernel(x, edge_index, W1, b1, W2, b2):
    n, d = x.shape
    e = edge_index.shape[1]

    epad = _ceil_to(e, _NC * _NS * 128)
    src_p = jnp.concatenate(
        [edge_index[0], jnp.zeros((epad - e,), jnp.int32)]).reshape(-1, 128)
    dst_p = jnp.concatenate(
        [edge_index[1], jnp.full((epad - e,), n, jnp.int32)]).reshape(-1, 128)

    nh = _ceil_to(n + 1, _NS * 128)

    deg_parts = _sc_degree(dst_p, nh)
    p0 = deg_parts[0, :n].reshape(n, 1)
    p1 = deg_parts[1, :n].reshape(n, 1)

    grid, rb = 10, n // 10
    b1r = b1.reshape(1, d)
    b2r = b2.reshape(1, d)

    h1, g1a, g1b = _tc_layer1(x, W1, p0, p1, grid, rb)
    agg1a, agg1b = _sc_aggregate(g1a, g1b, src_p, dst_p, nh)
    h2, g2a, g2b = _tc_layer2(agg1a[:n], agg1b[:n], h1, p0, p1, b1r, W2,
                              grid, rb)
    agg2a, agg2b = _sc_aggregate(g2a, g2b, src_p, dst_p, nh)
    out = _tc_final(agg2a[:n], agg2b[:n], h2, p0, p1, b2r, grid, rb)
    return out


# speedup vs baseline: 7.6378x; 1.0026x over previous
import dataclasses
import functools

import jax
import jax.numpy as jnp
from jax import lax
from jax.experimental import pallas as pl
from jax.experimental.pallas import tpu as pltpu
from jax.experimental.pallas import tpu_sc as plsc

_NC = 2
_NS = 16
_LANES = 16


def _sc_compiler_params():
    cp = pltpu.CompilerParams()
    if "needs_layout_passes" in pltpu.CompilerParams.__dataclass_fields__:
        cp = dataclasses.replace(cp, needs_layout_passes=False)
    return cp


def _ceil_to(a, m):
    return (a + m - 1) // m * m


def _sc_degree(dst2d, nh):
    rows = dst2d.shape[0]
    half = rows // _NC
    per_tile = half // _NS
    sl = nh // _NS
    mesh = plsc.VectorSubcoreMesh(core_axis_name="c", subcore_axis_name="s")

    @functools.partial(
        pl.kernel,
        out_type=jax.ShapeDtypeStruct((_NC, nh), jnp.float32),
        mesh=mesh,
        scratch_types=[
            pltpu.VMEM((per_tile, 128), jnp.int32),
            pltpu.VMEM((nh,), jnp.float32),
            pltpu.VMEM((sl,), jnp.float32),
            pltpu.VMEM((sl,), jnp.float32),
            pltpu.VMEM_SHARED((_NS, nh), jnp.float32),
        ],
        compiler_params=_sc_compiler_params(),
    )
    def k(dst_hbm, deg_hbm, dst_v, hist_v, acc_v, buf_v, part_sp):
        c = lax.axis_index("c")
        s = lax.axis_index("s")
        zero16 = jnp.zeros((_LANES,), jnp.float32)
        one16 = jnp.ones((_LANES,), jnp.float32)

        @pl.loop(0, nh, step=_LANES)
        def _(i):
            hist_v[pl.ds(pl.multiple_of(i, _LANES), _LANES)] = zero16

        pltpu.sync_copy(dst_hbm.at[pl.ds(c * half + s * per_tile, per_tile)],
                        dst_v)

        @pl.loop(0, per_tile)
        def _(r):
            for j in range(128 // _LANES):
                idx = dst_v[r, pl.ds(j * _LANES, _LANES)]
                plsc.addupdate_scatter(hist_v, [idx], one16)

        pltpu.sync_copy(hist_v, part_sp.at[s])
        plsc.subcore_barrier()

        pltpu.sync_copy(part_sp.at[0, pl.ds(s * sl, sl)], acc_v)
        for t in range(1, _NS):
            pltpu.sync_copy(part_sp.at[t, pl.ds(s * sl, sl)], buf_v)

            @pl.loop(0, sl, step=_LANES)
            def _(j):
                jj = pl.multiple_of(j, _LANES)
                acc_v[pl.ds(jj, _LANES)] = (
                    acc_v[pl.ds(jj, _LANES)] + buf_v[pl.ds(jj, _LANES)])

        @pl.when(c == 0)
        def _():
            pltpu.sync_copy(acc_v, deg_hbm.at[0, pl.ds(s * sl, sl)])

        @pl.when(c == 1)
        def _():
            pltpu.sync_copy(acc_v, deg_hbm.at[1, pl.ds(s * sl, sl)])

    return k(dst2d)


def _sc_aggregate(g0, g1, src2d, dst2d, nh):
    rows = src2d.shape[0]
    per_tile = rows // _NS
    nch = 2
    chunk = per_tile // nch
    zrows = nh // _NS
    mesh = plsc.VectorSubcoreMesh(core_axis_name="c", subcore_axis_name="s")

    @functools.partial(
        pl.kernel,
        out_type=[jax.ShapeDtypeStruct((nh, 128), jnp.float32)] * 2,
        mesh=mesh,
        scratch_types=[
            pltpu.VMEM((chunk, 128), jnp.int32),
            pltpu.VMEM((chunk, 128), jnp.int32),
            pltpu.VMEM((64, 128), jnp.float32),
            pltpu.VMEM((64, 128), jnp.float32),
            pltpu.VMEM((64, 128), jnp.float32),
            pltpu.VMEM((64, 128), jnp.float32),
            pltpu.VMEM_SHARED((nh, 128), jnp.float32),
            pltpu.SemaphoreType.DMA((4,)),
            pltpu.SemaphoreType.DMA((4,)),
        ],
    )
    def k(g0_hbm, g1_hbm, src_hbm, dst_hbm, out0_hbm, out1_hbm,
          src_v, dst_v, b0, b1, b2, b3, acc_sp, gsem, ssem):
        c = lax.axis_index("c")
        s = lax.axis_index("s")
        bufs = (b0, b1, b2, b3)
        zero16 = jnp.zeros((_LANES,), jnp.float32)

        @pl.loop(0, 64)
        def _(i):
            for j in range(128 // _LANES):
                b0[i, pl.ds(j * _LANES, _LANES)] = zero16

        for j in range(zrows // 64):
            pltpu.sync_copy(b0, acc_sp.at[pl.ds(s * zrows + j * 64, 64)])

        plsc.subcore_barrier()

        def run_half(g_hbm):
            for ch in range(nch):
                base = s * per_tile + ch * chunk
                pltpu.sync_copy(src_hbm.at[pl.ds(base, chunk)], src_v)
                pltpu.sync_copy(dst_hbm.at[pl.ds(base, chunk)], dst_v)

                @pl.loop(0, chunk // 2)
                def _(t):
                    k0 = 2 * t
                    cps = [
                        pltpu.async_copy(
                            g_hbm.at[src_v.at[k0 + i // 2,
                                              pl.ds((i % 2) * 64, 64)]],
                            bufs[i], gsem.at[i])
                        for i in range(4)
                    ]
                    scs = []
                    for i in range(4):
                        cps[i].wait()
                        scs.append(pltpu.async_copy(
                            bufs[i],
                            acc_sp.at[dst_v.at[k0 + i // 2,
                                               pl.ds((i % 2) * 64, 64)]],
                            ssem.at[i], add=True))
                    for sc in scs:
                        sc.wait()

        @pl.when(c == 0)
        def _():
            run_half(g0_hbm)

        @pl.when(c == 1)
        def _():
            run_half(g1_hbm)

        plsc.subcore_barrier()
        out_slc = pl.ds(s * zrows, zrows)

        @pl.when(c == 0)
        def _():
            pltpu.sync_copy(acc_sp.at[out_slc], out0_hbm.at[out_slc])

        @pl.when(c == 1)
        def _():
            pltpu.sync_copy(acc_sp.at[out_slc], out1_hbm.at[out_slc])

    return k(g0, g1, src2d, dst2d)


def _tc_layer1(x, w, p0, p1, grid, rb):
    n, d = x.shape

    def body(x_ref, w_ref, p0_ref, p1_ref, h_ref, g0_ref, g1_ref):
        h = jnp.dot(x_ref[...], w_ref[...],
                    preferred_element_type=jnp.float32,
                    precision=lax.Precision.HIGHEST)
        deg = p0_ref[...] + p1_ref[...] + 1.0
        dis = lax.rsqrt(deg)
        g = h * dis
        h_ref[...] = h
        g0_ref[...] = g[:, :d // 2]
        g1_ref[...] = g[:, d // 2:]

    return pl.pallas_call(
        body,
        grid=(grid,),
        in_specs=[
            pl.BlockSpec((rb, d), lambda i: (i, 0)),
            pl.BlockSpec((d, d), lambda i: (0, 0)),
            pl.BlockSpec((rb, 1), lambda i: (i, 0)),
            pl.BlockSpec((rb, 1), lambda i: (i, 0)),
        ],
        out_specs=[
            pl.BlockSpec((rb, d), lambda i: (i, 0)),
            pl.BlockSpec((rb, d // 2), lambda i: (i, 0)),
            pl.BlockSpec((rb, d // 2), lambda i: (i, 0)),
        ],
        out_shape=[
            jax.ShapeDtypeStruct((n, d), jnp.float32),
            jax.ShapeDtypeStruct((n, d // 2), jnp.float32),
            jax.ShapeDtypeStruct((n, d // 2), jnp.float32),
        ],
    )(x, w, p0, p1)


def _tc_layer2(a0, a1, h_prev, p0, p1, b, w, grid, rb):
    n = h_prev.shape[0]
    d = h_prev.shape[1]

    def body(a0_ref, a1_ref, h_ref, p0_ref, p1_ref, b_ref, w_ref,
             h2_ref, g0_ref, g1_ref):
        deg = p0_ref[...] + p1_ref[...] + 1.0
        dis = lax.rsqrt(deg)
        inv = 1.0 / deg
        agg = jnp.concatenate([a0_ref[...], a1_ref[...]], axis=1)
        act = jnp.maximum(agg * dis + h_ref[...] * inv + b_ref[...], 0.0)
        h2 = jnp.dot(act, w_ref[...],
                     preferred_element_type=jnp.float32,
                     precision=lax.Precision.HIGHEST)
        g2 = h2 * dis
        h2_ref[...] = h2
        g0_ref[...] = g2[:, :d // 2]
        g1_ref[...] = g2[:, d // 2:]

    return pl.pallas_call(
        body,
        grid=(grid,),
        in_specs=[
            pl.BlockSpec((rb, d // 2), lambda i: (i, 0)),
            pl.BlockSpec((rb, d // 2), lambda i: (i, 0)),
            pl.BlockSpec((rb, d), lambda i: (i, 0)),
            pl.BlockSpec((rb, 1), lambda i: (i, 0)),
            pl.BlockSpec((rb, 1), lambda i: (i, 0)),
            pl.BlockSpec((1, d), lambda i: (0, 0)),
            pl.BlockSpec((d, d), lambda i: (0, 0)),
        ],
        out_specs=[
            pl.BlockSpec((rb, d), lambda i: (i, 0)),
            pl.BlockSpec((rb, d // 2), lambda i: (i, 0)),
            pl.BlockSpec((rb, d // 2), lambda i: (i, 0)),
        ],
        out_shape=[
            jax.ShapeDtypeStruct((n, d), jnp.float32),
            jax.ShapeDtypeStruct((n, d // 2), jnp.float32),
            jax.ShapeDtypeStruct((n, d // 2), jnp.float32),
        ],
    )(a0, a1, h_prev, p0, p1, b, w)


def _tc_final(a0, a1, h_prev, p0, p1, b, grid, rb):
    n = h_prev.shape[0]
    d = h_prev.shape[1]

    def body(a0_ref, a1_ref, h_ref, p0_ref, p1_ref, b_ref, o_ref):
        deg = p0_ref[...] + p1_ref[...] + 1.0
        dis = lax.rsqrt(deg)
        inv = 1.0 / deg
        agg = jnp.concatenate([a0_ref[...], a1_ref[...]], axis=1)
        o_ref[...] = agg * dis + h_ref[...] * inv + b_ref[...]

    return pl.pallas_call(
        body,
        grid=(grid,),
        in_specs=[
            pl.BlockSpec((rb, d // 2), lambda i: (i, 0)),
            pl.BlockSpec((rb, d // 2), lambda i: (i, 0)),
            pl.BlockSpec((rb, d), lambda i: (i, 0)),
            pl.BlockSpec((rb, 1), lambda i: (i, 0)),
            pl.BlockSpec((rb, 1), lambda i: (i, 0)),
            pl.BlockSpec((1, d), lambda i: (0, 0)),
        ],
        out_specs=pl.BlockSpec((rb, d), lambda i: (i, 0)),
        out_shape=jax.ShapeDtypeStruct((n, d), jnp.float32),
    )(a0, a1, h_prev, p0, p1, b)


@jax.jit
def kernel(x, edge_index, W1, b1, W2, b2):
    n, d = x.shape
    e = edge_index.shape[1]

    epad = _ceil_to(e, _NC * _NS * 128)
    src_p = jnp.concatenate(
        [edge_index[0], jnp.zeros((epad - e,), jnp.int32)]).reshape(-1, 128)
    dst_p = jnp.concatenate(
        [edge_index[1], jnp.full((epad - e,), n, jnp.int32)]).reshape(-1, 128)

    nh = _ceil_to(n + 1, _NS * 128)

    deg_parts = _sc_degree(dst_p, nh)
    p0 = deg_parts[0, :n].reshape(n, 1)
    p1 = deg_parts[1, :n].reshape(n, 1)

    grid, rb = 10, n // 10
    b1r = b1.reshape(1, d)
    b2r = b2.reshape(1, d)

    h1, g1a, g1b = _tc_layer1(x, W1, p0, p1, grid, rb)
    agg1a, agg1b = _sc_aggregate(g1a, g1b, src_p, dst_p, nh)
    h2, g2a, g2b = _tc_layer2(agg1a[:n], agg1b[:n], h1, p0, p1, b1r, W2,
                              grid, rb)
    agg2a, agg2b = _sc_aggregate(g2a, g2b, src_p, dst_p, nh)
    out = _tc_final(agg2a[:n], agg2b[:n], h2, p0, p1, b2r, grid, rb)
    return out
